# Initial kernel scaffold; baseline (speedup 1.0000x reference)
import jax
import jax.numpy as jnp
from jax.experimental import pallas as pl


def kernel(node_x, edge_attr, edge_index, node_type, node_y, output_mask, params):
    raise NotImplementedError("write your pallas kernel here")


# speedup vs baseline: 2.8478x; 2.8478x over previous
import functools

import jax
import jax.numpy as jnp
from jax import lax
from jax.experimental import pallas as pl
from jax.experimental.pallas import tpu as pltpu
from jax.experimental.pallas import tpu_sc as plsc

N_NODES = 10000
N_EDGES = 320000
H = 128
HALF = 64
MP_NUM = 5
INFLOW, OUTFLOW, WALL_BOUNDARY, IN_WALL = 4, 5, 6, 7

NC = 2
NS = 16
NW = NC * NS
EDGES_PER_W = N_EDGES // NW
CHUNK = 80
N_CHUNKS = EDGES_PER_W // CHUNK
NPAD = 10240
STRIPE = NPAD // NS

F32 = jnp.float32


def _silu(x):
    return x * jax.lax.logistic(x)


def _layernorm(h, g, b):
    mu = jnp.mean(h, axis=-1, keepdims=True)
    var = jnp.mean((h - mu) * (h - mu), axis=-1, keepdims=True)
    return (h - mu) * jax.lax.rsqrt(var + 1e-5) * g + b


_sc_mesh = plsc.VectorSubcoreMesh(core_axis_name="c", subcore_axis_name="s")


@functools.partial(
    pl.kernel,
    mesh=_sc_mesh,
    out_type=(
        jax.ShapeDtypeStruct((N_EDGES, H), F32),
        jax.ShapeDtypeStruct((N_EDGES, H), F32),
    ),
    scratch_types=[
        pltpu.VMEM((CHUNK,), jnp.int32),
        pltpu.VMEM((CHUNK,), jnp.int32),
        pltpu.VMEM((CHUNK, H), F32),
        pltpu.VMEM((CHUNK, H), F32),
        pltpu.SemaphoreType.DMA,
    ],
)
def _sc_gather(ys_hbm, yr_hbm, s_hbm, r_hbm, gs_hbm, gr_hbm,
               sidx_v, ridx_v, rows_s, rows_r, sem):
    wid = lax.axis_index("s") * NC + lax.axis_index("c")
    base0 = wid * EDGES_PER_W

    def chunk(i, carry):
        base = base0 + i * CHUNK
        pltpu.sync_copy(s_hbm.at[pl.ds(base, CHUNK)], sidx_v)
        pltpu.sync_copy(r_hbm.at[pl.ds(base, CHUNK)], ridx_v)
        a = pltpu.async_copy(ys_hbm.at[sidx_v], rows_s, sem)
        b = pltpu.async_copy(yr_hbm.at[ridx_v], rows_r, sem)
        a.wait()
        b.wait()
        pltpu.sync_copy(rows_s, gs_hbm.at[pl.ds(base, CHUNK)])
        pltpu.sync_copy(rows_r, gr_hbm.at[pl.ds(base, CHUNK)])
        return carry

    lax.fori_loop(0, N_CHUNKS, chunk, 0)


@functools.partial(
    pl.kernel,
    mesh=_sc_mesh,
    out_type=jax.ShapeDtypeStruct((NC, NPAD, HALF), F32),
    scratch_types=[
        pltpu.VMEM((CHUNK,), jnp.int32),
        pltpu.VMEM((CHUNK,), jnp.int32),
        pltpu.VMEM((CHUNK, HALF), F32),
        pltpu.VMEM((CHUNK, HALF), F32),
        pltpu.VMEM_SHARED((NPAD, HALF), F32),
    ],
)
def _sc_scatter(el_hbm, er_hbm, r_hbm, s_hbm, zeros_hbm, out_hbm,
                ridx_v, sidx_v, buf_l, buf_r, acc_sh):
    cid = lax.axis_index("c")
    sid = lax.axis_index("s")
    pltpu.sync_copy(zeros_hbm.at[pl.ds(sid * STRIPE, STRIPE)],
                    acc_sh.at[pl.ds(sid * STRIPE, STRIPE)])
    plsc.subcore_barrier()

    wid = sid * NC + cid
    base0 = wid * EDGES_PER_W

    def chunk(i, carry):
        base = base0 + i * CHUNK
        pltpu.sync_copy(r_hbm.at[pl.ds(base, CHUNK)], ridx_v)
        pltpu.sync_copy(s_hbm.at[pl.ds(base, CHUNK)], sidx_v)
        pltpu.sync_copy(el_hbm.at[pl.ds(base, CHUNK)], buf_l)
        pltpu.sync_copy(er_hbm.at[pl.ds(base, CHUNK)], buf_r)
        pltpu.sync_copy(buf_l, acc_sh.at[ridx_v], add=True)
        pltpu.sync_copy(buf_r, acc_sh.at[sidx_v], add=True)
        return carry

    lax.fori_loop(0, N_CHUNKS, chunk, 0)
    plsc.subcore_barrier()
    pltpu.sync_copy(acc_sh.at[pl.ds(sid * STRIPE, STRIPE)],
                    out_hbm.at[cid, pl.ds(sid * STRIPE, STRIPE)])


_full = lambda shp: pl.BlockSpec(shp, lambda i: (0,) * len(shp))


def _row_spec(rows, cols):
    return pl.BlockSpec((rows, cols), lambda i: (i, 0))


def _mlp3(h, w1_ref, b1_ref, w2_ref, b2_ref, w3_ref, b3_ref):
    h = _silu(h + b1_ref[...])
    h = _silu(jnp.dot(h, w2_ref[...], preferred_element_type=F32) + b2_ref[...])
    return jnp.dot(h, w3_ref[...], preferred_element_type=F32) + b3_ref[...]


def _edge_body(e_ref, gs_ref, gr_ref,
               w1_ref, b1_ref, w2_ref, b2_ref, w3_ref, b3_ref, g_ref, be_ref,
               el_ref, er_ref, eout_ref):
    e = e_ref[...]
    h = jnp.dot(e, w1_ref[...], preferred_element_type=F32) + gs_ref[...] + gr_ref[...]
    h = _mlp3(h, w1_ref, b1_ref, w2_ref, b2_ref, w3_ref, b3_ref)
    h = _layernorm(h, g_ref[...], be_ref[...])
    el_ref[...] = h[:, :HALF]
    er_ref[...] = h[:, HALF:]
    eout_ref[...] = h + e


def _edge_enc_body(ea_ref,
                   w1_ref, b1_ref, w2_ref, b2_ref, w3_ref, b3_ref, g_ref, be_ref,
                   eout_ref):
    h = jnp.dot(ea_ref[...], w1_ref[...], preferred_element_type=F32)
    h = _mlp3(h, w1_ref, b1_ref, w2_ref, b2_ref, w3_ref, b3_ref)
    eout_ref[...] = _layernorm(h, g_ref[...], be_ref[...])


def _node_enc_body(nx_ref,
                   w1_ref, b1_ref, w2_ref, b2_ref, w3_ref, b3_ref, g_ref, be_ref,
                   wys_ref, wyr_ref,
                   x_ref, ys_ref, yr_ref):
    h = jnp.dot(nx_ref[...], w1_ref[...], preferred_element_type=F32)
    h = _mlp3(h, w1_ref, b1_ref, w2_ref, b2_ref, w3_ref, b3_ref)
    x = _layernorm(h, g_ref[...], be_ref[...])
    x_ref[...] = x
    ys_ref[...] = jnp.dot(x, wys_ref[...], preferred_element_type=F32)
    yr_ref[...] = jnp.dot(x, wyr_ref[...], preferred_element_type=F32)


def _node_body(x_ref, a0_ref, a1_ref,
               w1x_ref, w1a_ref, b1_ref, w2_ref, b2_ref, w3_ref, b3_ref,
               g_ref, be_ref, wys_ref, wyr_ref,
               x_out_ref, ys_ref, yr_ref):
    x = x_ref[...]
    agg = a0_ref[...] + a1_ref[...]
    h = (jnp.dot(x, w1x_ref[...], preferred_element_type=F32)
         + jnp.dot(agg, w1a_ref[...], preferred_element_type=F32))
    h = _mlp3(h, w1x_ref, b1_ref, w2_ref, b2_ref, w3_ref, b3_ref)
    x_new = _layernorm(h, g_ref[...], be_ref[...]) + x
    x_out_ref[...] = x_new
    ys_ref[...] = jnp.dot(x_new, wys_ref[...], preferred_element_type=F32)
    yr_ref[...] = jnp.dot(x_new, wyr_ref[...], preferred_element_type=F32)


def _node_last_body(x_ref, a0_ref, a1_ref,
                    w1x_ref, w1a_ref, b1_ref, w2_ref, b2_ref, w3_ref, b3_ref,
                    g_ref, be_ref,
                    x_out_ref):
    x = x_ref[...]
    agg = a0_ref[...] + a1_ref[...]
    h = (jnp.dot(x, w1x_ref[...], preferred_element_type=F32)
         + jnp.dot(agg, w1a_ref[...], preferred_element_type=F32))
    h = _mlp3(h, w1x_ref, b1_ref, w2_ref, b2_ref, w3_ref, b3_ref)
    x_out_ref[...] = _layernorm(h, g_ref[...], be_ref[...]) + x


def _dec_body(x_ref, emb_ref,
              w1x_ref, w1e_ref, b1_ref, w2_ref, b2_ref, w3_ref, b3_ref,
              out_ref):
    h = (jnp.dot(x_ref[...], w1x_ref[...], preferred_element_type=F32)
         + jnp.dot(emb_ref[...], w1e_ref[...], preferred_element_type=F32))
    h = _silu(h + b1_ref[...])
    h = _silu(jnp.dot(h, w2_ref[...], preferred_element_type=F32) + b2_ref[...])
    out_ref[...] = jnp.dot(h, w3_ref[...], preferred_element_type=F32) + b3_ref[...]


ER = 4000
NR = 2000

_tc_params = pltpu.CompilerParams(dimension_semantics=("arbitrary",))


def _edge_mlp(e, gs, gr, w1, b1, w2, b2, w3, b3, g, be):
    wspec = [_full((H, H)), _full((1, H)), _full((H, H)), _full((1, H)),
             _full((H, H)), _full((1, H)), _full((1, H)), _full((1, H))]
    return pl.pallas_call(
        _edge_body,
        grid=(N_EDGES // ER,),
        in_specs=[_row_spec(ER, H)] * 3 + wspec,
        out_specs=(_row_spec(ER, HALF), _row_spec(ER, HALF), _row_spec(ER, H)),
        out_shape=(jax.ShapeDtypeStruct((N_EDGES, HALF), F32),
                   jax.ShapeDtypeStruct((N_EDGES, HALF), F32),
                   jax.ShapeDtypeStruct((N_EDGES, H), F32)),
        compiler_params=_tc_params,
    )(e, gs, gr, w1, b1, w2, b2, w3, b3, g, be)


def _edge_encoder(ea, w1, b1, w2, b2, w3, b3, g, be):
    wspec = [_full((H, H)), _full((1, H)), _full((H, H)), _full((1, H)),
             _full((H, H)), _full((1, H)), _full((1, H)), _full((1, H))]
    return pl.pallas_call(
        _edge_enc_body,
        grid=(N_EDGES // ER,),
        in_specs=[_row_spec(ER, H)] + wspec,
        out_specs=_row_spec(ER, H),
        out_shape=jax.ShapeDtypeStruct((N_EDGES, H), F32),
        compiler_params=_tc_params,
    )(ea, w1, b1, w2, b2, w3, b3, g, be)


def _node_encoder(nx, w1, b1, w2, b2, w3, b3, g, be, wys, wyr):
    wspec = [_full((H, H)), _full((1, H)), _full((H, H)), _full((1, H)),
             _full((H, H)), _full((1, H)), _full((1, H)), _full((1, H)),
             _full((H, H)), _full((H, H))]
    return pl.pallas_call(
        _node_enc_body,
        grid=(N_NODES // NR,),
        in_specs=[_row_spec(NR, H)] + wspec,
        out_specs=(_row_spec(NR, H), _row_spec(NR, H), _row_spec(NR, H)),
        out_shape=(jax.ShapeDtypeStruct((N_NODES, H), F32),
                   jax.ShapeDtypeStruct((N_NODES, H), F32),
                   jax.ShapeDtypeStruct((N_NODES, H), F32)),
        compiler_params=_tc_params,
    )(nx, w1, b1, w2, b2, w3, b3, g, be, wys, wyr)


def _node_mlp(x, a0, a1, w1x, w1a, b1, w2, b2, w3, b3, g, be, wys, wyr):
    wspec = [_full((H, H)), _full((HALF, H)), _full((1, H)), _full((H, H)),
             _full((1, H)), _full((H, H)), _full((1, H)), _full((1, H)),
             _full((1, H)), _full((H, H)), _full((H, H))]
    return pl.pallas_call(
        _node_body,
        grid=(N_NODES // NR,),
        in_specs=[_row_spec(NR, H), _row_spec(NR, HALF), _row_spec(NR, HALF)] + wspec,
        out_specs=(_row_spec(NR, H), _row_spec(NR, H), _row_spec(NR, H)),
        out_shape=(jax.ShapeDtypeStruct((N_NODES, H), F32),
                   jax.ShapeDtypeStruct((N_NODES, H), F32),
                   jax.ShapeDtypeStruct((N_NODES, H), F32)),
        compiler_params=_tc_params,
    )(x, a0, a1, w1x, w1a, b1, w2, b2, w3, b3, g, be, wys, wyr)


def _node_mlp_last(x, a0, a1, w1x, w1a, b1, w2, b2, w3, b3, g, be):
    wspec = [_full((H, H)), _full((HALF, H)), _full((1, H)), _full((H, H)),
             _full((1, H)), _full((H, H)), _full((1, H)), _full((1, H)),
             _full((1, H))]
    return pl.pallas_call(
        _node_last_body,
        grid=(N_NODES // NR,),
        in_specs=[_row_spec(NR, H), _row_spec(NR, HALF), _row_spec(NR, HALF)] + wspec,
        out_specs=_row_spec(NR, H),
        out_shape=jax.ShapeDtypeStruct((N_NODES, H), F32),
        compiler_params=_tc_params,
    )(x, a0, a1, w1x, w1a, b1, w2, b2, w3, b3, g, be)


def _decoder(x, emb, w1x, w1e, b1, w2, b2, w3, b3):
    wspec = [_full((H, H)), _full((H, H)), _full((1, H)), _full((H, H)),
             _full((1, H)), _full((H, H)), _full((1, H))]
    return pl.pallas_call(
        _dec_body,
        grid=(N_NODES // NR,),
        in_specs=[_row_spec(NR, H), _row_spec(NR, H)] + wspec,
        out_specs=_row_spec(NR, H),
        out_shape=jax.ShapeDtypeStruct((N_NODES, H), F32),
        compiler_params=_tc_params,
    )(x, emb, w1x, w1e, b1, w2, b2, w3, b3)


def _lin(p, i):
    w, b = p["lins"][i]
    return w, b.reshape(1, H)


def kernel(node_x, edge_attr, edge_index, node_type, node_y, output_mask, params):
    senders = edge_index[0]
    receivers = edge_index[1]

    blocks = params["blocks"]
    eb_w1 = [blk["eb"]["lins"][0][0] for blk in blocks]
    wys = [w[H:2 * H] for w in eb_w1]
    wyr = [w[2 * H:] for w in eb_w1]

    enc_nb, enc_eb = params["enc_nb"], params["enc_eb"]
    nw1, nb1 = _lin(enc_nb, 0)
    nw2, nb2 = _lin(enc_nb, 1)
    nw3, nb3 = _lin(enc_nb, 2)
    ng, nbe = (v.reshape(1, H) for v in enc_nb["ln"])
    x, ys, yr = _node_encoder(node_x, nw1, nb1, nw2, nb2, nw3, nb3, ng, nbe,
                              wys[0], wyr[0])

    ew1, eb1 = _lin(enc_eb, 0)
    ew2, eb2 = _lin(enc_eb, 1)
    ew3, eb3 = _lin(enc_eb, 2)
    eg, ebe = (v.reshape(1, H) for v in enc_eb["ln"])
    e = _edge_encoder(edge_attr, ew1, eb1, ew2, eb2, ew3, eb3, eg, ebe)

    emb = x
    zeros_pad = jnp.zeros((NPAD, HALF), F32)

    for i, blk in enumerate(blocks):
        gs, gr = _sc_gather(ys, yr, senders, receivers)

        ebp = blk["eb"]
        w1e = eb_w1[i][:H]
        _, b1 = _lin(ebp, 0)
        w2, b2 = _lin(ebp, 1)
        w3, b3 = _lin(ebp, 2)
        g, be = (v.reshape(1, H) for v in ebp["ln"])
        el, er, e = _edge_mlp(e, gs, gr, w1e, b1, w2, b2, w3, b3, g, be)

        aggp = _sc_scatter(el, er, receivers, senders, zeros_pad)

        nbp = blk["nb"]
        nw1f = nbp["lins"][0][0]
        w1x, w1a = nw1f[:H], nbp["lins"][0][0][H:]
        _, b1n = _lin(nbp, 0)
        w2n, b2n = _lin(nbp, 1)
        w3n, b3n = _lin(nbp, 2)
        gn, ben = (v.reshape(1, H) for v in nbp["ln"])
        a0 = aggp[0]
        a1 = aggp[1]
        if i + 1 < MP_NUM:
            x, ys, yr = _node_mlp(x, a0, a1, w1x, w1a, b1n, w2n, b2n, w3n,
                                  b3n, gn, ben, wys[i + 1], wyr[i + 1])
        else:
            x = _node_mlp_last(x, a0, a1, w1x, w1a, b1n, w2n, b2n, w3n, b3n,
                               gn, ben)

    dec_p = params["dec"]
    dw1 = dec_p["lins"][0][0]
    dw1x, dw1e = dw1[:H], dw1[H:]
    _, db1 = _lin(dec_p, 0)
    dw2, db2 = _lin(dec_p, 1)
    dw3_raw, db3_raw = dec_p["lins"][2]
    dw3 = jnp.zeros((H, H), F32).at[:, :3].set(dw3_raw)
    db3 = jnp.zeros((1, H), F32).at[0, :3].set(db3_raw)
    dec = _decoder(x, emb, dw1x, dw1e, db1, dw2, db2, dw3, db3)[:, :3]

    uv = 10.0 * jnp.tanh(dec[:, 0:2] / 10.0)
    p = 10.0 * jnp.tanh(dec[:, 2:3] / 10.0)
    nt = node_type[:, 0]
    boundary_fluid_mask = (nt == INFLOW) | (nt == WALL_BOUNDARY) | (nt == IN_WALL)
    uv = jnp.where(boundary_fluid_mask[:, None], node_y[:, 0:2], uv)
    p = jnp.where((nt == OUTFLOW)[:, None], 0.0, p)
    p = jnp.where((nt == IN_WALL)[:, None], 0.0, p)
    return jnp.concatenate([uv, p], axis=1) * output_mask


# speedup vs baseline: 3.3427x; 1.1738x over previous
import functools

import jax
import jax.numpy as jnp
from jax import lax
from jax.experimental import pallas as pl
from jax.experimental.pallas import tpu as pltpu
from jax.experimental.pallas import tpu_sc as plsc

N_NODES = 10000
N_EDGES = 320000
H = 128
HALF = 64
MP_NUM = 5
INFLOW, OUTFLOW, WALL_BOUNDARY, IN_WALL = 4, 5, 6, 7

NC = 2
NS = 16
NW = NC * NS
EDGES_PER_W = N_EDGES // NW
CHUNK = 80
N_CHUNKS = EDGES_PER_W // CHUNK
NPAD = 10240
STRIPE = NPAD // NS

F32 = jnp.float32


def _silu(x):
    return x * jax.lax.logistic(x)


def _layernorm(h, g, b):
    mu = jnp.mean(h, axis=-1, keepdims=True)
    var = jnp.mean((h - mu) * (h - mu), axis=-1, keepdims=True)
    return (h - mu) * jax.lax.rsqrt(var + 1e-5) * g + b


_sc_mesh = plsc.VectorSubcoreMesh(core_axis_name="c", subcore_axis_name="s")

DEPTH = 4
N_GROUPS = N_CHUNKS // DEPTH
CHUNKS_PER_W = N_CHUNKS


@functools.partial(
    pl.kernel,
    mesh=_sc_mesh,
    out_type=(
        jax.ShapeDtypeStruct((N_EDGES, H), F32),
        jax.ShapeDtypeStruct((N_EDGES, H), F32),
    ),
    scratch_types=(
        [pltpu.VMEM((CHUNK,), jnp.int32)] * (2 * DEPTH)
        + [pltpu.VMEM((CHUNK, H), F32)] * (2 * DEPTH)
        + [pltpu.SemaphoreType.DMA] * (2 * DEPTH)
    ),
)
def _sc_gather(ys_hbm, yr_hbm, s_hbm, r_hbm, gs_hbm, gr_hbm, *scr):
    sidx = scr[0:DEPTH]
    ridx = scr[DEPTH:2 * DEPTH]
    rs = scr[2 * DEPTH:3 * DEPTH]
    rr = scr[3 * DEPTH:4 * DEPTH]
    si = scr[4 * DEPTH:5 * DEPTH]
    sg = scr[5 * DEPTH:6 * DEPTH]

    wid = lax.axis_index("s") * NC + lax.axis_index("c")
    base0 = wid * EDGES_PER_W

    def issue_idx(c, b):
        return (
            pltpu.async_copy(s_hbm.at[pl.ds(base0 + c * CHUNK, CHUNK)], sidx[b], si[b]),
            pltpu.async_copy(r_hbm.at[pl.ds(base0 + c * CHUNK, CHUNK)], ridx[b], si[b]),
        )

    def issue_gather(b):
        return (
            pltpu.async_copy(ys_hbm.at[sidx[b]], rs[b], sg[b]),
            pltpu.async_copy(yr_hbm.at[ridx[b]], rr[b], sg[b]),
        )

    def issue_write(c, b):
        return (
            pltpu.async_copy(rs[b], gs_hbm.at[pl.ds(base0 + c * CHUNK, CHUNK)], sg[b]),
            pltpu.async_copy(rr[b], gr_hbm.at[pl.ds(base0 + c * CHUNK, CHUNK)], sg[b]),
        )

    def group(c0):
        ids = [issue_idx(c0 + b, b) for b in range(DEPTH)]
        gds = []
        for b in range(DEPTH):
            ids[b][0].wait()
            ids[b][1].wait()
            gds.append(issue_gather(b))
        wds = []
        for b in range(DEPTH):
            gds[b][0].wait()
            gds[b][1].wait()
            wds.append(issue_write(c0 + b, b))
        for a, bb in wds:
            a.wait()
            bb.wait()

    def body(g, carry):
        group(g * DEPTH)
        return carry

    lax.fori_loop(0, N_GROUPS, body, 0)
    for c in range(N_GROUPS * DEPTH, CHUNKS_PER_W):
        group_tail = issue_idx(c, 0)
        group_tail[0].wait()
        group_tail[1].wait()
        a, bb = issue_gather(0)
        a.wait()
        bb.wait()
        a, bb = issue_write(c, 0)
        a.wait()
        bb.wait()


@functools.partial(
    pl.kernel,
    mesh=_sc_mesh,
    out_type=jax.ShapeDtypeStruct((NC, NPAD, HALF), F32),
    scratch_types=(
        [pltpu.VMEM((CHUNK,), jnp.int32)] * (2 * DEPTH)
        + [pltpu.VMEM((CHUNK, HALF), F32)] * (2 * DEPTH)
        + [pltpu.VMEM_SHARED((NPAD, HALF), F32)]
        + [pltpu.SemaphoreType.DMA] * (2 * DEPTH)
    ),
)
def _sc_scatter(el_hbm, er_hbm, r_hbm, s_hbm, zeros_hbm, out_hbm, *scr):
    ri = scr[0:DEPTH]
    si = scr[DEPTH:2 * DEPTH]
    bl = scr[2 * DEPTH:3 * DEPTH]
    br = scr[3 * DEPTH:4 * DEPTH]
    acc_sh = scr[4 * DEPTH]
    sl = scr[1 + 4 * DEPTH:1 + 5 * DEPTH]
    ss = scr[1 + 5 * DEPTH:1 + 6 * DEPTH]

    cid = lax.axis_index("c")
    sid = lax.axis_index("s")
    pltpu.sync_copy(zeros_hbm.at[pl.ds(sid * STRIPE, STRIPE)],
                    acc_sh.at[pl.ds(sid * STRIPE, STRIPE)])

    wid = sid * NC + cid
    base0 = wid * EDGES_PER_W
    plsc.subcore_barrier()

    def chunk(i, carry):
        base = base0 + i * CHUNK
        pltpu.sync_copy(r_hbm.at[pl.ds(base, CHUNK)], ri[0])
        pltpu.sync_copy(s_hbm.at[pl.ds(base, CHUNK)], si[0])
        pltpu.sync_copy(el_hbm.at[pl.ds(base, CHUNK)], bl[0])
        pltpu.sync_copy(er_hbm.at[pl.ds(base, CHUNK)], br[0])
        pltpu.sync_copy(bl[0], acc_sh.at[ri[0]], add=True)
        pltpu.sync_copy(br[0], acc_sh.at[si[0]], add=True)
        return carry

    lax.fori_loop(0, N_CHUNKS, chunk, 0)
    del sl, ss
    plsc.subcore_barrier()
    pltpu.sync_copy(acc_sh.at[pl.ds(sid * STRIPE, STRIPE)],
                    out_hbm.at[cid, pl.ds(sid * STRIPE, STRIPE)])


_full = lambda shp: pl.BlockSpec(shp, lambda i: (0,) * len(shp))


def _row_spec(rows, cols):
    return pl.BlockSpec((rows, cols), lambda i: (i, 0))


def _mlp3(h, w1_ref, b1_ref, w2_ref, b2_ref, w3_ref, b3_ref):
    h = _silu(h + b1_ref[...])
    h = _silu(jnp.dot(h, w2_ref[...], preferred_element_type=F32) + b2_ref[...])
    return jnp.dot(h, w3_ref[...], preferred_element_type=F32) + b3_ref[...]


def _edge_body(e_ref, gs_ref, gr_ref,
               w1_ref, b1_ref, w2_ref, b2_ref, w3_ref, b3_ref, g_ref, be_ref,
               el_ref, er_ref, eout_ref):
    e = e_ref[...]
    h = jnp.dot(e, w1_ref[...], preferred_element_type=F32) + gs_ref[...] + gr_ref[...]
    h = _mlp3(h, w1_ref, b1_ref, w2_ref, b2_ref, w3_ref, b3_ref)
    h = _layernorm(h, g_ref[...], be_ref[...])
    el_ref[...] = h[:, :HALF]
    er_ref[...] = h[:, HALF:]
    eout_ref[...] = h + e


def _edge_enc_body(ea_ref,
                   w1_ref, b1_ref, w2_ref, b2_ref, w3_ref, b3_ref, g_ref, be_ref,
                   eout_ref):
    h = jnp.dot(ea_ref[...], w1_ref[...], preferred_element_type=F32)
    h = _mlp3(h, w1_ref, b1_ref, w2_ref, b2_ref, w3_ref, b3_ref)
    eout_ref[...] = _layernorm(h, g_ref[...], be_ref[...])


def _node_enc_body(nx_ref,
                   w1_ref, b1_ref, w2_ref, b2_ref, w3_ref, b3_ref, g_ref, be_ref,
                   wys_ref, wyr_ref,
                   x_ref, ys_ref, yr_ref):
    h = jnp.dot(nx_ref[...], w1_ref[...], preferred_element_type=F32)
    h = _mlp3(h, w1_ref, b1_ref, w2_ref, b2_ref, w3_ref, b3_ref)
    x = _layernorm(h, g_ref[...], be_ref[...])
    x_ref[...] = x
    ys_ref[...] = jnp.dot(x, wys_ref[...], preferred_element_type=F32)
    yr_ref[...] = jnp.dot(x, wyr_ref[...], preferred_element_type=F32)


def _node_body(x_ref, a0_ref, a1_ref,
               w1x_ref, w1a_ref, b1_ref, w2_ref, b2_ref, w3_ref, b3_ref,
               g_ref, be_ref, wys_ref, wyr_ref,
               x_out_ref, ys_ref, yr_ref):
    x = x_ref[...]
    agg = a0_ref[...] + a1_ref[...]
    h = (jnp.dot(x, w1x_ref[...], preferred_element_type=F32)
         + jnp.dot(agg, w1a_ref[...], preferred_element_type=F32))
    h = _mlp3(h, w1x_ref, b1_ref, w2_ref, b2_ref, w3_ref, b3_ref)
    x_new = _layernorm(h, g_ref[...], be_ref[...]) + x
    x_out_ref[...] = x_new
    ys_ref[...] = jnp.dot(x_new, wys_ref[...], preferred_element_type=F32)
    yr_ref[...] = jnp.dot(x_new, wyr_ref[...], preferred_element_type=F32)


def _node_last_body(x_ref, a0_ref, a1_ref,
                    w1x_ref, w1a_ref, b1_ref, w2_ref, b2_ref, w3_ref, b3_ref,
                    g_ref, be_ref,
                    x_out_ref):
    x = x_ref[...]
    agg = a0_ref[...] + a1_ref[...]
    h = (jnp.dot(x, w1x_ref[...], preferred_element_type=F32)
         + jnp.dot(agg, w1a_ref[...], preferred_element_type=F32))
    h = _mlp3(h, w1x_ref, b1_ref, w2_ref, b2_ref, w3_ref, b3_ref)
    x_out_ref[...] = _layernorm(h, g_ref[...], be_ref[...]) + x


def _dec_body(x_ref, emb_ref,
              w1x_ref, w1e_ref, b1_ref, w2_ref, b2_ref, w3_ref, b3_ref,
              out_ref):
    h = (jnp.dot(x_ref[...], w1x_ref[...], preferred_element_type=F32)
         + jnp.dot(emb_ref[...], w1e_ref[...], preferred_element_type=F32))
    h = _silu(h + b1_ref[...])
    h = _silu(jnp.dot(h, w2_ref[...], preferred_element_type=F32) + b2_ref[...])
    out_ref[...] = jnp.dot(h, w3_ref[...], preferred_element_type=F32) + b3_ref[...]


ER = 4000
NR = 2000

_tc_params = pltpu.CompilerParams(dimension_semantics=("arbitrary",))


def _edge_mlp(e, gs, gr, w1, b1, w2, b2, w3, b3, g, be):
    wspec = [_full((H, H)), _full((1, H)), _full((H, H)), _full((1, H)),
             _full((H, H)), _full((1, H)), _full((1, H)), _full((1, H))]
    return pl.pallas_call(
        _edge_body,
        grid=(N_EDGES // ER,),
        in_specs=[_row_spec(ER, H)] * 3 + wspec,
        out_specs=(_row_spec(ER, HALF), _row_spec(ER, HALF), _row_spec(ER, H)),
        out_shape=(jax.ShapeDtypeStruct((N_EDGES, HALF), F32),
                   jax.ShapeDtypeStruct((N_EDGES, HALF), F32),
                   jax.ShapeDtypeStruct((N_EDGES, H), F32)),
        compiler_params=_tc_params,
    )(e, gs, gr, w1, b1, w2, b2, w3, b3, g, be)


def _edge_encoder(ea, w1, b1, w2, b2, w3, b3, g, be):
    wspec = [_full((H, H)), _full((1, H)), _full((H, H)), _full((1, H)),
             _full((H, H)), _full((1, H)), _full((1, H)), _full((1, H))]
    return pl.pallas_call(
        _edge_enc_body,
        grid=(N_EDGES // ER,),
        in_specs=[_row_spec(ER, H)] + wspec,
        out_specs=_row_spec(ER, H),
        out_shape=jax.ShapeDtypeStruct((N_EDGES, H), F32),
        compiler_params=_tc_params,
    )(ea, w1, b1, w2, b2, w3, b3, g, be)


def _node_encoder(nx, w1, b1, w2, b2, w3, b3, g, be, wys, wyr):
    wspec = [_full((H, H)), _full((1, H)), _full((H, H)), _full((1, H)),
             _full((H, H)), _full((1, H)), _full((1, H)), _full((1, H)),
             _full((H, H)), _full((H, H))]
    return pl.pallas_call(
        _node_enc_body,
        grid=(N_NODES // NR,),
        in_specs=[_row_spec(NR, H)] + wspec,
        out_specs=(_row_spec(NR, H), _row_spec(NR, H), _row_spec(NR, H)),
        out_shape=(jax.ShapeDtypeStruct((N_NODES, H), F32),
                   jax.ShapeDtypeStruct((N_NODES, H), F32),
                   jax.ShapeDtypeStruct((N_NODES, H), F32)),
        compiler_params=_tc_params,
    )(nx, w1, b1, w2, b2, w3, b3, g, be, wys, wyr)


def _node_mlp(x, a0, a1, w1x, w1a, b1, w2, b2, w3, b3, g, be, wys, wyr):
    wspec = [_full((H, H)), _full((HALF, H)), _full((1, H)), _full((H, H)),
             _full((1, H)), _full((H, H)), _full((1, H)), _full((1, H)),
             _full((1, H)), _full((H, H)), _full((H, H))]
    return pl.pallas_call(
        _node_body,
        grid=(N_NODES // NR,),
        in_specs=[_row_spec(NR, H), _row_spec(NR, HALF), _row_spec(NR, HALF)] + wspec,
        out_specs=(_row_spec(NR, H), _row_spec(NR, H), _row_spec(NR, H)),
        out_shape=(jax.ShapeDtypeStruct((N_NODES, H), F32),
                   jax.ShapeDtypeStruct((N_NODES, H), F32),
                   jax.ShapeDtypeStruct((N_NODES, H), F32)),
        compiler_params=_tc_params,
    )(x, a0, a1, w1x, w1a, b1, w2, b2, w3, b3, g, be, wys, wyr)


def _node_mlp_last(x, a0, a1, w1x, w1a, b1, w2, b2, w3, b3, g, be):
    wspec = [_full((H, H)), _full((HALF, H)), _full((1, H)), _full((H, H)),
             _full((1, H)), _full((H, H)), _full((1, H)), _full((1, H)),
             _full((1, H))]
    return pl.pallas_call(
        _node_last_body,
        grid=(N_NODES // NR,),
        in_specs=[_row_spec(NR, H), _row_spec(NR, HALF), _row_spec(NR, HALF)] + wspec,
        out_specs=_row_spec(NR, H),
        out_shape=jax.ShapeDtypeStruct((N_NODES, H), F32),
        compiler_params=_tc_params,
    )(x, a0, a1, w1x, w1a, b1, w2, b2, w3, b3, g, be)


def _decoder(x, emb, w1x, w1e, b1, w2, b2, w3, b3):
    wspec = [_full((H, H)), _full((H, H)), _full((1, H)), _full((H, H)),
             _full((1, H)), _full((H, H)), _full((1, H))]
    return pl.pallas_call(
        _dec_body,
        grid=(N_NODES // NR,),
        in_specs=[_row_spec(NR, H), _row_spec(NR, H)] + wspec,
        out_specs=_row_spec(NR, H),
        out_shape=jax.ShapeDtypeStruct((N_NODES, H), F32),
        compiler_params=_tc_params,
    )(x, emb, w1x, w1e, b1, w2, b2, w3, b3)


def _lin(p, i):
    w, b = p["lins"][i]
    return w, b.reshape(1, H)


def kernel(node_x, edge_attr, edge_index, node_type, node_y, output_mask, params):
    senders = edge_index[0]
    receivers = edge_index[1]

    blocks = params["blocks"]
    eb_w1 = [blk["eb"]["lins"][0][0] for blk in blocks]
    wys = [w[H:2 * H] for w in eb_w1]
    wyr = [w[2 * H:] for w in eb_w1]

    enc_nb, enc_eb = params["enc_nb"], params["enc_eb"]
    nw1, nb1 = _lin(enc_nb, 0)
    nw2, nb2 = _lin(enc_nb, 1)
    nw3, nb3 = _lin(enc_nb, 2)
    ng, nbe = (v.reshape(1, H) for v in enc_nb["ln"])
    x, ys, yr = _node_encoder(node_x, nw1, nb1, nw2, nb2, nw3, nb3, ng, nbe,
                              wys[0], wyr[0])

    ew1, eb1 = _lin(enc_eb, 0)
    ew2, eb2 = _lin(enc_eb, 1)
    ew3, eb3 = _lin(enc_eb, 2)
    eg, ebe = (v.reshape(1, H) for v in enc_eb["ln"])
    e = _edge_encoder(edge_attr, ew1, eb1, ew2, eb2, ew3, eb3, eg, ebe)

    emb = x
    zeros_pad = jnp.zeros((NPAD, HALF), F32)

    for i, blk in enumerate(blocks):
        gs, gr = _sc_gather(ys, yr, senders, receivers)

        ebp = blk["eb"]
        w1e = eb_w1[i][:H]
        _, b1 = _lin(ebp, 0)
        w2, b2 = _lin(ebp, 1)
        w3, b3 = _lin(ebp, 2)
        g, be = (v.reshape(1, H) for v in ebp["ln"])
        el, er, e = _edge_mlp(e, gs, gr, w1e, b1, w2, b2, w3, b3, g, be)

        aggp = _sc_scatter(el, er, receivers, senders, zeros_pad)

        nbp = blk["nb"]
        nw1f = nbp["lins"][0][0]
        w1x, w1a = nw1f[:H], nbp["lins"][0][0][H:]
        _, b1n = _lin(nbp, 0)
        w2n, b2n = _lin(nbp, 1)
        w3n, b3n = _lin(nbp, 2)
        gn, ben = (v.reshape(1, H) for v in nbp["ln"])
        a0 = aggp[0]
        a1 = aggp[1]
        if i + 1 < MP_NUM:
            x, ys, yr = _node_mlp(x, a0, a1, w1x, w1a, b1n, w2n, b2n, w3n,
                                  b3n, gn, ben, wys[i + 1], wyr[i + 1])
        else:
            x = _node_mlp_last(x, a0, a1, w1x, w1a, b1n, w2n, b2n, w3n, b3n,
                               gn, ben)

    dec_p = params["dec"]
    dw1 = dec_p["lins"][0][0]
    dw1x, dw1e = dw1[:H], dw1[H:]
    _, db1 = _lin(dec_p, 0)
    dw2, db2 = _lin(dec_p, 1)
    dw3_raw, db3_raw = dec_p["lins"][2]
    dw3 = jnp.zeros((H, H), F32).at[:, :3].set(dw3_raw)
    db3 = jnp.zeros((1, H), F32).at[0, :3].set(db3_raw)
    dec = _decoder(x, emb, dw1x, dw1e, db1, dw2, db2, dw3, db3)[:, :3]

    uv = 10.0 * jnp.tanh(dec[:, 0:2] / 10.0)
    p = 10.0 * jnp.tanh(dec[:, 2:3] / 10.0)
    nt = node_type[:, 0]
    boundary_fluid_mask = (nt == INFLOW) | (nt == WALL_BOUNDARY) | (nt == IN_WALL)
    uv = jnp.where(boundary_fluid_mask[:, None], node_y[:, 0:2], uv)
    p = jnp.where((nt == OUTFLOW)[:, None], 0.0, p)
    p = jnp.where((nt == IN_WALL)[:, None], 0.0, p)
    return jnp.concatenate([uv, p], axis=1) * output_mask


# speedup vs baseline: 4.1732x; 1.2485x over previous
import functools

import jax
import jax.numpy as jnp
from jax import lax
from jax.experimental import pallas as pl
from jax.experimental.pallas import tpu as pltpu
from jax.experimental.pallas import tpu_sc as plsc

N_NODES = 10000
N_EDGES = 320000
H = 128
HALF = 64
MP_NUM = 5
INFLOW, OUTFLOW, WALL_BOUNDARY, IN_WALL = 4, 5, 6, 7

NC = 2
NS = 16
NW = NC * NS
EDGES_PER_W = N_EDGES // NW
CHUNK = 80
N_CHUNKS = EDGES_PER_W // CHUNK
NPAD = 10240
STRIPE = NPAD // NS

F32 = jnp.float32


def _silu(x):
    return x * jax.lax.logistic(x)


def _layernorm(h, g, b):
    mu = jnp.mean(h, axis=-1, keepdims=True)
    var = jnp.mean((h - mu) * (h - mu), axis=-1, keepdims=True)
    return (h - mu) * jax.lax.rsqrt(var + 1e-5) * g + b


_sc_mesh = plsc.VectorSubcoreMesh(core_axis_name="c", subcore_axis_name="s")

DEPTH = 4
N_GROUPS = N_CHUNKS // DEPTH
CHUNKS_PER_W = N_CHUNKS

SUP = 128
SUB = 128
K_SUB = SUP // SUB
N_SUP = EDGES_PER_W // SUP
TAIL = EDGES_PER_W - N_SUP * SUP


@functools.partial(
    pl.kernel,
    mesh=_sc_mesh,
    out_type=(
        jax.ShapeDtypeStruct((N_EDGES, H), F32),
        jax.ShapeDtypeStruct((N_EDGES, H), F32),
    ),
    scratch_types=(
        [pltpu.VMEM((CHUNK,), jnp.int32)] * (2 * DEPTH)
        + [pltpu.VMEM((CHUNK, H), F32)] * (2 * DEPTH)
        + [pltpu.SemaphoreType.DMA] * (2 * DEPTH)
    ),
)
def _sc_gather(ys_hbm, yr_hbm, s_hbm, r_hbm, gs_hbm, gr_hbm, *scr):
    sidx = scr[0:DEPTH]
    ridx = scr[DEPTH:2 * DEPTH]
    rs = scr[2 * DEPTH:3 * DEPTH]
    rr = scr[3 * DEPTH:4 * DEPTH]
    si = scr[4 * DEPTH:5 * DEPTH]
    sg = scr[5 * DEPTH:6 * DEPTH]

    wid = lax.axis_index("s") * NC + lax.axis_index("c")
    base0 = wid * EDGES_PER_W

    def issue_idx(c, b):
        return (
            pltpu.async_copy(s_hbm.at[pl.ds(base0 + c * CHUNK, CHUNK)], sidx[b], si[b]),
            pltpu.async_copy(r_hbm.at[pl.ds(base0 + c * CHUNK, CHUNK)], ridx[b], si[b]),
        )

    def issue_gather(b):
        return (
            pltpu.async_copy(ys_hbm.at[sidx[b]], rs[b], sg[b]),
            pltpu.async_copy(yr_hbm.at[ridx[b]], rr[b], sg[b]),
        )

    def issue_write(c, b):
        return (
            pltpu.async_copy(rs[b], gs_hbm.at[pl.ds(base0 + c * CHUNK, CHUNK)], sg[b]),
            pltpu.async_copy(rr[b], gr_hbm.at[pl.ds(base0 + c * CHUNK, CHUNK)], sg[b]),
        )

    def group(c0):
        ids = [issue_idx(c0 + b, b) for b in range(DEPTH)]
        gds = []
        for b in range(DEPTH):
            ids[b][0].wait()
            ids[b][1].wait()
            gds.append(issue_gather(b))
        wds = []
        for b in range(DEPTH):
            gds[b][0].wait()
            gds[b][1].wait()
            wds.append(issue_write(c0 + b, b))
        for a, bb in wds:
            a.wait()
            bb.wait()

    def body(g, carry):
        group(g * DEPTH)
        return carry

    lax.fori_loop(0, N_GROUPS, body, 0)
    for c in range(N_GROUPS * DEPTH, CHUNKS_PER_W):
        group_tail = issue_idx(c, 0)
        group_tail[0].wait()
        group_tail[1].wait()
        a, bb = issue_gather(0)
        a.wait()
        bb.wait()
        a, bb = issue_write(c, 0)
        a.wait()
        bb.wait()


@functools.partial(
    pl.kernel,
    mesh=_sc_mesh,
    out_type=jax.ShapeDtypeStruct((NC, NPAD, HALF), F32),
    scratch_types=(
        [pltpu.VMEM((SUB,), jnp.int32)] * (2 * K_SUB)
        + [pltpu.VMEM((TAIL,), jnp.int32)] * 2
        + [pltpu.VMEM((SUP, HALF), F32)] * 2
        + [pltpu.VMEM_SHARED((NPAD, HALF), F32)]
        + [pltpu.SemaphoreType.DMA]
    ),
)
def _sc_scatter(el_hbm, er_hbm, r_hbm, s_hbm, zeros_hbm, out_hbm, *scr):
    ri = scr[0:K_SUB]
    si = scr[K_SUB:2 * K_SUB]
    rt, st = scr[2 * K_SUB], scr[2 * K_SUB + 1]
    bl, br = scr[2 * K_SUB + 2], scr[2 * K_SUB + 3]
    acc_sh = scr[2 * K_SUB + 4]
    sem = scr[2 * K_SUB + 5]

    cid = lax.axis_index("c")
    sid = lax.axis_index("s")
    pltpu.sync_copy(zeros_hbm.at[pl.ds(sid * STRIPE, STRIPE)],
                    acc_sh.at[pl.ds(sid * STRIPE, STRIPE)])

    wid = sid * NC + cid
    base0 = wid * EDGES_PER_W
    plsc.subcore_barrier()

    def super_chunk(base):
        ds_ = [
            pltpu.async_copy(el_hbm.at[pl.ds(base, SUP)], bl, sem),
            pltpu.async_copy(er_hbm.at[pl.ds(base, SUP)], br, sem),
        ]
        for k in range(K_SUB):
            ds_.append(pltpu.async_copy(r_hbm.at[pl.ds(base + k * SUB, SUB)], ri[k], sem))
            ds_.append(pltpu.async_copy(s_hbm.at[pl.ds(base + k * SUB, SUB)], si[k], sem))
        for d in ds_:
            d.wait()
        for k in range(K_SUB):
            pltpu.sync_copy(bl.at[pl.ds(k * SUB, SUB)], acc_sh.at[ri[k]], add=True)
            pltpu.sync_copy(br.at[pl.ds(k * SUB, SUB)], acc_sh.at[si[k]], add=True)

    def body(g, carry):
        super_chunk(base0 + g * SUP)
        return carry

    lax.fori_loop(0, N_SUP, body, 0)

    base = base0 + N_SUP * SUP
    ds_ = [
        pltpu.async_copy(el_hbm.at[pl.ds(base, TAIL)], bl.at[pl.ds(0, TAIL)], sem),
        pltpu.async_copy(er_hbm.at[pl.ds(base, TAIL)], br.at[pl.ds(0, TAIL)], sem),
        pltpu.async_copy(r_hbm.at[pl.ds(base, TAIL)], rt, sem),
        pltpu.async_copy(s_hbm.at[pl.ds(base, TAIL)], st, sem),
    ]
    for d in ds_:
        d.wait()
    pltpu.sync_copy(bl.at[pl.ds(0, TAIL)], acc_sh.at[rt], add=True)
    pltpu.sync_copy(br.at[pl.ds(0, TAIL)], acc_sh.at[st], add=True)

    plsc.subcore_barrier()
    pltpu.sync_copy(acc_sh.at[pl.ds(sid * STRIPE, STRIPE)],
                    out_hbm.at[cid, pl.ds(sid * STRIPE, STRIPE)])


_full = lambda shp: pl.BlockSpec(shp, lambda i: (0,) * len(shp))


def _row_spec(rows, cols):
    return pl.BlockSpec((rows, cols), lambda i: (i, 0))


def _mlp3(h, w1_ref, b1_ref, w2_ref, b2_ref, w3_ref, b3_ref):
    h = _silu(h + b1_ref[...])
    h = _silu(jnp.dot(h, w2_ref[...], preferred_element_type=F32) + b2_ref[...])
    return jnp.dot(h, w3_ref[...], preferred_element_type=F32) + b3_ref[...]


def _edge_body(e_ref, gs_ref, gr_ref,
               w1_ref, b1_ref, w2_ref, b2_ref, w3_ref, b3_ref, g_ref, be_ref,
               el_ref, er_ref, eout_ref):
    e = e_ref[...]
    h = jnp.dot(e, w1_ref[...], preferred_element_type=F32) + gs_ref[...] + gr_ref[...]
    h = _mlp3(h, w1_ref, b1_ref, w2_ref, b2_ref, w3_ref, b3_ref)
    h = _layernorm(h, g_ref[...], be_ref[...])
    el_ref[...] = h[:, :HALF]
    er_ref[...] = h[:, HALF:]
    eout_ref[...] = h + e


def _edge_enc_body(ea_ref,
                   w1_ref, b1_ref, w2_ref, b2_ref, w3_ref, b3_ref, g_ref, be_ref,
                   eout_ref):
    h = jnp.dot(ea_ref[...], w1_ref[...], preferred_element_type=F32)
    h = _mlp3(h, w1_ref, b1_ref, w2_ref, b2_ref, w3_ref, b3_ref)
    eout_ref[...] = _layernorm(h, g_ref[...], be_ref[...])


def _node_enc_body(nx_ref,
                   w1_ref, b1_ref, w2_ref, b2_ref, w3_ref, b3_ref, g_ref, be_ref,
                   wys_ref, wyr_ref,
                   x_ref, ys_ref, yr_ref):
    h = jnp.dot(nx_ref[...], w1_ref[...], preferred_element_type=F32)
    h = _mlp3(h, w1_ref, b1_ref, w2_ref, b2_ref, w3_ref, b3_ref)
    x = _layernorm(h, g_ref[...], be_ref[...])
    x_ref[...] = x
    ys_ref[...] = jnp.dot(x, wys_ref[...], preferred_element_type=F32)
    yr_ref[...] = jnp.dot(x, wyr_ref[...], preferred_element_type=F32)


def _node_body(x_ref, a0_ref, a1_ref,
               w1x_ref, w1a_ref, b1_ref, w2_ref, b2_ref, w3_ref, b3_ref,
               g_ref, be_ref, wys_ref, wyr_ref,
               x_out_ref, ys_ref, yr_ref):
    x = x_ref[...]
    agg = a0_ref[...] + a1_ref[...]
    h = (jnp.dot(x, w1x_ref[...], preferred_element_type=F32)
         + jnp.dot(agg, w1a_ref[...], preferred_element_type=F32))
    h = _mlp3(h, w1x_ref, b1_ref, w2_ref, b2_ref, w3_ref, b3_ref)
    x_new = _layernorm(h, g_ref[...], be_ref[...]) + x
    x_out_ref[...] = x_new
    ys_ref[...] = jnp.dot(x_new, wys_ref[...], preferred_element_type=F32)
    yr_ref[...] = jnp.dot(x_new, wyr_ref[...], preferred_element_type=F32)


def _node_last_body(x_ref, a0_ref, a1_ref,
                    w1x_ref, w1a_ref, b1_ref, w2_ref, b2_ref, w3_ref, b3_ref,
                    g_ref, be_ref,
                    x_out_ref):
    x = x_ref[...]
    agg = a0_ref[...] + a1_ref[...]
    h = (jnp.dot(x, w1x_ref[...], preferred_element_type=F32)
         + jnp.dot(agg, w1a_ref[...], preferred_element_type=F32))
    h = _mlp3(h, w1x_ref, b1_ref, w2_ref, b2_ref, w3_ref, b3_ref)
    x_out_ref[...] = _layernorm(h, g_ref[...], be_ref[...]) + x


def _dec_body(x_ref, emb_ref,
              w1x_ref, w1e_ref, b1_ref, w2_ref, b2_ref, w3_ref, b3_ref,
              out_ref):
    h = (jnp.dot(x_ref[...], w1x_ref[...], preferred_element_type=F32)
         + jnp.dot(emb_ref[...], w1e_ref[...], preferred_element_type=F32))
    h = _silu(h + b1_ref[...])
    h = _silu(jnp.dot(h, w2_ref[...], preferred_element_type=F32) + b2_ref[...])
    out_ref[...] = jnp.dot(h, w3_ref[...], preferred_element_type=F32) + b3_ref[...]


ER = 4000
NR = 2000

_tc_params = pltpu.CompilerParams(dimension_semantics=("arbitrary",))


def _edge_mlp(e, gs, gr, w1, b1, w2, b2, w3, b3, g, be):
    wspec = [_full((H, H)), _full((1, H)), _full((H, H)), _full((1, H)),
             _full((H, H)), _full((1, H)), _full((1, H)), _full((1, H))]
    return pl.pallas_call(
        _edge_body,
        grid=(N_EDGES // ER,),
        in_specs=[_row_spec(ER, H)] * 3 + wspec,
        out_specs=(_row_spec(ER, HALF), _row_spec(ER, HALF), _row_spec(ER, H)),
        out_shape=(jax.ShapeDtypeStruct((N_EDGES, HALF), F32),
                   jax.ShapeDtypeStruct((N_EDGES, HALF), F32),
                   jax.ShapeDtypeStruct((N_EDGES, H), F32)),
        compiler_params=_tc_params,
    )(e, gs, gr, w1, b1, w2, b2, w3, b3, g, be)


def _edge_encoder(ea, w1, b1, w2, b2, w3, b3, g, be):
    wspec = [_full((H, H)), _full((1, H)), _full((H, H)), _full((1, H)),
             _full((H, H)), _full((1, H)), _full((1, H)), _full((1, H))]
    return pl.pallas_call(
        _edge_enc_body,
        grid=(N_EDGES // ER,),
        in_specs=[_row_spec(ER, H)] + wspec,
        out_specs=_row_spec(ER, H),
        out_shape=jax.ShapeDtypeStruct((N_EDGES, H), F32),
        compiler_params=_tc_params,
    )(ea, w1, b1, w2, b2, w3, b3, g, be)


def _node_encoder(nx, w1, b1, w2, b2, w3, b3, g, be, wys, wyr):
    wspec = [_full((H, H)), _full((1, H)), _full((H, H)), _full((1, H)),
             _full((H, H)), _full((1, H)), _full((1, H)), _full((1, H)),
             _full((H, H)), _full((H, H))]
    return pl.pallas_call(
        _node_enc_body,
        grid=(N_NODES // NR,),
        in_specs=[_row_spec(NR, H)] + wspec,
        out_specs=(_row_spec(NR, H), _row_spec(NR, H), _row_spec(NR, H)),
        out_shape=(jax.ShapeDtypeStruct((N_NODES, H), F32),
                   jax.ShapeDtypeStruct((N_NODES, H), F32),
                   jax.ShapeDtypeStruct((N_NODES, H), F32)),
        compiler_params=_tc_params,
    )(nx, w1, b1, w2, b2, w3, b3, g, be, wys, wyr)


def _node_mlp(x, a0, a1, w1x, w1a, b1, w2, b2, w3, b3, g, be, wys, wyr):
    wspec = [_full((H, H)), _full((HALF, H)), _full((1, H)), _full((H, H)),
             _full((1, H)), _full((H, H)), _full((1, H)), _full((1, H)),
             _full((1, H)), _full((H, H)), _full((H, H))]
    return pl.pallas_call(
        _node_body,
        grid=(N_NODES // NR,),
        in_specs=[_row_spec(NR, H), _row_spec(NR, HALF), _row_spec(NR, HALF)] + wspec,
        out_specs=(_row_spec(NR, H), _row_spec(NR, H), _row_spec(NR, H)),
        out_shape=(jax.ShapeDtypeStruct((N_NODES, H), F32),
                   jax.ShapeDtypeStruct((N_NODES, H), F32),
                   jax.ShapeDtypeStruct((N_NODES, H), F32)),
        compiler_params=_tc_params,
    )(x, a0, a1, w1x, w1a, b1, w2, b2, w3, b3, g, be, wys, wyr)


def _node_mlp_last(x, a0, a1, w1x, w1a, b1, w2, b2, w3, b3, g, be):
    wspec = [_full((H, H)), _full((HALF, H)), _full((1, H)), _full((H, H)),
             _full((1, H)), _full((H, H)), _full((1, H)), _full((1, H)),
             _full((1, H))]
    return pl.pallas_call(
        _node_last_body,
        grid=(N_NODES // NR,),
        in_specs=[_row_spec(NR, H), _row_spec(NR, HALF), _row_spec(NR, HALF)] + wspec,
        out_specs=_row_spec(NR, H),
        out_shape=jax.ShapeDtypeStruct((N_NODES, H), F32),
        compiler_params=_tc_params,
    )(x, a0, a1, w1x, w1a, b1, w2, b2, w3, b3, g, be)


def _decoder(x, emb, w1x, w1e, b1, w2, b2, w3, b3):
    wspec = [_full((H, H)), _full((H, H)), _full((1, H)), _full((H, H)),
             _full((1, H)), _full((H, H)), _full((1, H))]
    return pl.pallas_call(
        _dec_body,
        grid=(N_NODES // NR,),
        in_specs=[_row_spec(NR, H), _row_spec(NR, H)] + wspec,
        out_specs=_row_spec(NR, H),
        out_shape=jax.ShapeDtypeStruct((N_NODES, H), F32),
        compiler_params=_tc_params,
    )(x, emb, w1x, w1e, b1, w2, b2, w3, b3)


def _lin(p, i):
    w, b = p["lins"][i]
    return w, b.reshape(1, H)


def kernel(node_x, edge_attr, edge_index, node_type, node_y, output_mask, params):
    senders = edge_index[0]
    receivers = edge_index[1]

    blocks = params["blocks"]
    eb_w1 = [blk["eb"]["lins"][0][0] for blk in blocks]
    wys = [w[H:2 * H] for w in eb_w1]
    wyr = [w[2 * H:] for w in eb_w1]

    enc_nb, enc_eb = params["enc_nb"], params["enc_eb"]
    nw1, nb1 = _lin(enc_nb, 0)
    nw2, nb2 = _lin(enc_nb, 1)
    nw3, nb3 = _lin(enc_nb, 2)
    ng, nbe = (v.reshape(1, H) for v in enc_nb["ln"])
    x, ys, yr = _node_encoder(node_x, nw1, nb1, nw2, nb2, nw3, nb3, ng, nbe,
                              wys[0], wyr[0])

    ew1, eb1 = _lin(enc_eb, 0)
    ew2, eb2 = _lin(enc_eb, 1)
    ew3, eb3 = _lin(enc_eb, 2)
    eg, ebe = (v.reshape(1, H) for v in enc_eb["ln"])
    e = _edge_encoder(edge_attr, ew1, eb1, ew2, eb2, ew3, eb3, eg, ebe)

    emb = x
    zeros_pad = jnp.zeros((NPAD, HALF), F32)

    for i, blk in enumerate(blocks):
        gs, gr = _sc_gather(ys, yr, senders, receivers)

        ebp = blk["eb"]
        w1e = eb_w1[i][:H]
        _, b1 = _lin(ebp, 0)
        w2, b2 = _lin(ebp, 1)
        w3, b3 = _lin(ebp, 2)
        g, be = (v.reshape(1, H) for v in ebp["ln"])
        el, er, e = _edge_mlp(e, gs, gr, w1e, b1, w2, b2, w3, b3, g, be)

        aggp = _sc_scatter(el, er, receivers, senders, zeros_pad)

        nbp = blk["nb"]
        nw1f = nbp["lins"][0][0]
        w1x, w1a = nw1f[:H], nbp["lins"][0][0][H:]
        _, b1n = _lin(nbp, 0)
        w2n, b2n = _lin(nbp, 1)
        w3n, b3n = _lin(nbp, 2)
        gn, ben = (v.reshape(1, H) for v in nbp["ln"])
        a0 = aggp[0]
        a1 = aggp[1]
        if i + 1 < MP_NUM:
            x, ys, yr = _node_mlp(x, a0, a1, w1x, w1a, b1n, w2n, b2n, w3n,
                                  b3n, gn, ben, wys[i + 1], wyr[i + 1])
        else:
            x = _node_mlp_last(x, a0, a1, w1x, w1a, b1n, w2n, b2n, w3n, b3n,
                               gn, ben)

    dec_p = params["dec"]
    dw1 = dec_p["lins"][0][0]
    dw1x, dw1e = dw1[:H], dw1[H:]
    _, db1 = _lin(dec_p, 0)
    dw2, db2 = _lin(dec_p, 1)
    dw3_raw, db3_raw = dec_p["lins"][2]
    dw3 = jnp.zeros((H, H), F32).at[:, :3].set(dw3_raw)
    db3 = jnp.zeros((1, H), F32).at[0, :3].set(db3_raw)
    dec = _decoder(x, emb, dw1x, dw1e, db1, dw2, db2, dw3, db3)[:, :3]

    uv = 10.0 * jnp.tanh(dec[:, 0:2] / 10.0)
    p = 10.0 * jnp.tanh(dec[:, 2:3] / 10.0)
    nt = node_type[:, 0]
    boundary_fluid_mask = (nt == INFLOW) | (nt == WALL_BOUNDARY) | (nt == IN_WALL)
    uv = jnp.where(boundary_fluid_mask[:, None], node_y[:, 0:2], uv)
    p = jnp.where((nt == OUTFLOW)[:, None], 0.0, p)
    p = jnp.where((nt == IN_WALL)[:, None], 0.0, p)
    return jnp.concatenate([uv, p], axis=1) * output_mask
